# baseline (speedup 1.0000x reference)
import functools

import jax
import jax.numpy as jnp
import numpy as np
from jax import lax
from jax.experimental import pallas as pl
from jax.experimental.pallas import tpu as pltpu
from jax.experimental.pallas import tpu_sc as plsc

_NUM_PER_CLS = np.array([5000,4773,4556,4349,4151,3963,3782,3611,3447,3290,3141,2998,2862,2732,2608,2489,2376,2268,2165,2067,1973,1883,1798,1716,1638,1564,1493,1425,1360,1298,1239,1183,1129,1078,1029,982,937,895,854,815,778,743,709,677,646,617,589,562,536,512,489,466,445,425,406,387,370,353,337,321,307,293,280,267,255,243,232,222,212,202,193,184,176,168,160,153,146,139,133,127,121,116,110,105,101,96,92,88,84,80,76,73,70,66,63,60,58,55,52,50], dtype=np.float64)
_MAX_M = 0.5
_S = 30.0
_m = 1.0 / np.sqrt(np.sqrt(_NUM_PER_CLS))
_m = _m * (_MAX_M / np.max(_m))
_beta = 0.9999
_eff = 1.0 - np.power(_beta, _NUM_PER_CLS)
_w = (1.0 - _beta) / np.array(_eff)
_w = _w / np.sum(_w) * len(_NUM_PER_CLS)
_SMT_TBL = jnp.asarray(np.pad(_S * _m, (0, 28)), dtype=jnp.float32)
_W_TBL = jnp.asarray(np.pad(_w, (0, 28)), dtype=jnp.float32)


def _gather_body(rows_per_w, t_hbm, m_hbm, w_hbm, mt_out, wt_out,
                 t_v, mv_v, wv_v, sem):
    wid = lax.axis_index("s") * 2 + lax.axis_index("c")
    base = wid * rows_per_w
    nchunk = rows_per_w // 128
    for j in range(nchunk):
        pltpu.sync_copy(t_hbm.at[pl.ds(base + j * 128, 128)], t_v.at[j])
    cps = []
    for j in range(nchunk):
        cps.append(pltpu.async_copy(m_hbm.at[t_v.at[j]], mv_v.at[j], sem))
        cps.append(pltpu.async_copy(w_hbm.at[t_v.at[j]], wv_v.at[j], sem))
    for cp in cps:
        cp.wait()
    for j in range(nchunk):
        pltpu.sync_copy(mv_v.at[j], mt_out.at[pl.ds(base + j * 128, 128)])
        pltpu.sync_copy(wv_v.at[j], wt_out.at[pl.ds(base + j * 128, 128)])


def _sc_gather(target):
    b = target.shape[0]
    rows_per_w = b // 32
    nchunk = rows_per_w // 128
    f32 = jnp.float32
    mesh = plsc.VectorSubcoreMesh(core_axis_name="c", subcore_axis_name="s")
    return pl.kernel(
        functools.partial(_gather_body, rows_per_w),
        mesh=mesh,
        out_type=[jax.ShapeDtypeStruct((b,), f32)] * 2,
        scratch_types=[
            pltpu.VMEM((nchunk, 128), jnp.int32),
            pltpu.VMEM((nchunk, 128), f32),
            pltpu.VMEM((nchunk, 128), f32),
            pltpu.SemaphoreType.DMA,
        ],
    )(target, _SMT_TBL, _W_TBL)


def _dense_body(x_ref, t_ref, rmx_ref, sum0_ref, a_ref):
    x = x_ref[...]
    t = t_ref[...]
    bm, c = x.shape
    rowmax = jnp.max(x, axis=1, keepdims=True)
    e = jnp.exp(_S * (x - rowmax))
    sum0_ref[...] = jnp.sum(e, axis=1)
    rmx_ref[...] = rowmax[:, 0]
    j = lax.broadcasted_iota(jnp.int32, (bm, c), 1)
    a_ref[...] = jnp.sum(jnp.where(j == t, x, 0.0), axis=1)


def _combine_body(rmx_ref, sum0_ref, a_ref, smt_ref, wt_ref, out_ref):
    rowmax = _S * rmx_ref[...]
    sum0 = sum0_ref[...]
    a = _S * a_ref[...]
    smt = smt_ref[...]
    wt = wt_ref[...]
    et = jnp.exp(a - rowmax)
    sum_corr = jnp.maximum(sum0 - et, 0.0) + et * jnp.exp(-smt)
    ce = rowmax + jnp.log(sum_corr) - a + smt
    out_ref[0, 0] = jnp.sum(wt * ce) / jnp.sum(wt)


@jax.jit
def kernel(x, target):
    b, c = x.shape
    bm = 2048
    smt, wt = _sc_gather(target)
    rmx, sum0, a_raw = pl.pallas_call(
        _dense_body,
        grid=(b // bm,),
        in_specs=[
            pl.BlockSpec((bm, c), lambda i: (i, 0)),
            pl.BlockSpec((bm, 1), lambda i: (i, 0)),
        ],
        out_specs=[pl.BlockSpec((bm,), lambda i: (i,))] * 3,
        out_shape=[jax.ShapeDtypeStruct((b,), jnp.float32)] * 3,
        compiler_params=pltpu.CompilerParams(
            dimension_semantics=("arbitrary",),
        ),
    )(x, target.reshape(b, 1))
    out = pl.pallas_call(
        _combine_body,
        out_specs=pl.BlockSpec(memory_space=pltpu.SMEM),
        out_shape=jax.ShapeDtypeStruct((1, 1), jnp.float32),
    )(rmx, sum0, a_raw, smt, wt)
    return out[0, 0]
